# baseline (speedup 1.0000x reference)
import functools

import jax
import jax.numpy as jnp
from jax import lax
from jax.experimental import pallas as pl
from jax.experimental.pallas import tpu as pltpu
from jax.experimental.pallas import tpu_sc as plsc

VOCAB = 1000000
EMBED_DIM = 64
BATCH = 16384
SEQ = 200

NUM_CORES = 2
NUM_SUBCORES = 16
NUM_WORKERS = NUM_CORES * NUM_SUBCORES
BAGS_PER_WORKER = BATCH // NUM_WORKERS
LANES = 16
NUM_CHUNKS = -(-SEQ // LANES)
SEQ_PAD = NUM_CHUNKS * LANES


def _embed_mean_body(idx_hbm, table_hbm, out_hbm, idx_v, rows_v, out_v, sem):
    wid = lax.axis_index("s") * NUM_CORES + lax.axis_index("c")
    base = wid * BAGS_PER_WORKER
    inv_n = jnp.float32(1.0 / SEQ)

    for c in range(NUM_CHUNKS):
        idx_v[pl.ds(c * LANES, LANES)] = jnp.zeros((LANES,), jnp.int32)

    def bag_body(i, _):
        pltpu.sync_copy(idx_hbm.at[pl.ds((base + i) * SEQ, SEQ)],
                        idx_v.at[pl.ds(0, SEQ)])
        copies = []
        for c in range(NUM_CHUNKS):
            idx_reg = idx_v[pl.ds(c * LANES, LANES)]
            copies.append(
                pltpu.async_copy(
                    table_hbm.at[idx_reg],
                    rows_v.at[pl.ds(c * LANES, LANES)],
                    sem,
                )
            )
        for cp in copies:
            cp.wait()

        def acc_body(j, carry):
            return tuple(
                carry[c] + rows_v[j, pl.ds(c * 16, 16)] for c in range(4)
            )

        zeros = jnp.zeros((16,), jnp.float32)
        acc = lax.fori_loop(0, SEQ, acc_body, (zeros, zeros, zeros, zeros))
        for c in range(4):
            out_v[pl.ds(i * EMBED_DIM + c * 16, 16)] = acc[c] * inv_n
        return 0

    lax.fori_loop(0, BAGS_PER_WORKER, bag_body, 0)
    pltpu.sync_copy(out_v, out_hbm.at[pl.ds(base * EMBED_DIM,
                                            BAGS_PER_WORKER * EMBED_DIM)])


@jax.jit
def _embed_mean(input_ids, emb_table):
    mesh = plsc.VectorSubcoreMesh(
        core_axis_name="c", subcore_axis_name="s",
        num_cores=NUM_CORES, num_subcores=NUM_SUBCORES)
    out = pl.kernel(
        _embed_mean_body,
        out_type=jax.ShapeDtypeStruct((BATCH * EMBED_DIM,), jnp.float32),
        mesh=mesh,
        scratch_types=[
            pltpu.VMEM((SEQ_PAD,), jnp.int32),
            pltpu.VMEM((SEQ_PAD, EMBED_DIM), jnp.float32),
            pltpu.VMEM((BAGS_PER_WORKER * EMBED_DIM,), jnp.float32),
            pltpu.SemaphoreType.DMA,
        ],
        compiler_params=pltpu.CompilerParams(use_tc_tiling_on_sc=False),
    )(input_ids.reshape(-1), emb_table)
    return out.reshape(BATCH, EMBED_DIM)


MLP_BLOCK = 1024
OUT_PAD = 128


def _mlp_body(x_ref, w1_ref, b1_ref, w2_ref, b2_ref, wa_ref, ba_ref,
              wc_ref, bc_ref, out_ref):
    x = x_ref[...]
    h = jnp.maximum(jnp.dot(x, w1_ref[...],
                            preferred_element_type=jnp.float32) + b1_ref[...], 0.0)
    h = jnp.maximum(jnp.dot(h, w2_ref[...],
                            preferred_element_type=jnp.float32) + b2_ref[...], 0.0)
    gate = jax.nn.sigmoid(jnp.dot(h, wa_ref[...],
                                  preferred_element_type=jnp.float32) + ba_ref[...])
    h = h * gate
    out_ref[...] = jnp.dot(h, wc_ref[...],
                           preferred_element_type=jnp.float32) + bc_ref[...]


@jax.jit
def _mlp(x, W1, b1, W2, b2, Wa, ba, Wc, bc):
    h1 = W1.shape[1]
    h2 = W2.shape[1]
    ncls = Wc.shape[1]
    Wc_pad = jnp.zeros((h2, OUT_PAD), jnp.float32).at[:, :ncls].set(Wc)
    bc_pad = jnp.zeros((1, OUT_PAD), jnp.float32).at[0, :ncls].set(bc)
    grid = BATCH // MLP_BLOCK
    full = lambda shape: pl.BlockSpec(shape, lambda i: (0, 0))
    out = pl.pallas_call(
        _mlp_body,
        grid=(grid,),
        in_specs=[
            pl.BlockSpec((MLP_BLOCK, EMBED_DIM), lambda i: (i, 0)),
            full((EMBED_DIM, h1)),
            full((1, h1)),
            full((h1, h2)),
            full((1, h2)),
            full((h2, h2)),
            full((1, h2)),
            full((h2, OUT_PAD)),
            full((1, OUT_PAD)),
        ],
        out_specs=pl.BlockSpec((MLP_BLOCK, OUT_PAD), lambda i: (i, 0)),
        out_shape=jax.ShapeDtypeStruct((BATCH, OUT_PAD), jnp.float32),
    )(x, W1, b1.reshape(1, -1), W2, b2.reshape(1, -1),
      Wa, ba.reshape(1, -1), Wc_pad, bc_pad)
    return out[:, :ncls]


def kernel(input_ids, emb_table, W1, b1, W2, b2, Wa, ba, Wc, bc):
    embedded = _embed_mean(input_ids, emb_table)
    return _mlp(embedded, W1, b1, W2, b2, Wa, ba, Wc, bc)


# speedup vs baseline: 2.7997x; 2.7997x over previous
import functools

import jax
import jax.numpy as jnp
from jax import lax
from jax.experimental import pallas as pl
from jax.experimental.pallas import tpu as pltpu
from jax.experimental.pallas import tpu_sc as plsc

VOCAB = 1000000
EMBED_DIM = 64
BATCH = 16384
SEQ = 200

NUM_CORES = 2
NUM_SUBCORES = 16
NUM_WORKERS = NUM_CORES * NUM_SUBCORES
BAGS_PER_WORKER = BATCH // NUM_WORKERS
IBLK = 64
NUM_BLOCKS = BAGS_PER_WORKER // IBLK
CHUNK_A = 128
CHUNK_B = SEQ - CHUNK_A
ACC_UNROLL = 8


def _embed_mean_body(idx_hbm, table_hbm, out_hbm,
                     idx0, idx1, rows0, rows1, out_v,
                     isem0, isem1, gsem0, gsem1):
    wid = lax.axis_index("s") * NUM_CORES + lax.axis_index("c")
    base = wid * BAGS_PER_WORKER
    inv_n = jnp.float32(1.0 / SEQ)
    idx_bufs = (idx0, idx1)
    isems = (isem0, isem1)
    rows_bufs = (rows0, rows1)
    gsems = (gsem0, gsem1)

    def fire_idx(b):
        pltpu.async_copy(
            idx_hbm.at[pl.ds((base + b * IBLK) * SEQ, IBLK * SEQ)],
            idx_bufs[b % 2], isems[b % 2])

    def wait_idx(b):
        pltpu.make_async_copy(
            idx_hbm.at[pl.ds(0, IBLK * SEQ)], idx_bufs[b % 2],
            isems[b % 2]).wait()

    def fire_bag(b, jl, p):
        off = jl * SEQ
        ib = idx_bufs[b % 2]
        pltpu.async_copy(table_hbm.at[ib.at[pl.ds(off, CHUNK_A)]],
                         rows_bufs[p].at[pl.ds(0, CHUNK_A)], gsems[p])
        pltpu.async_copy(table_hbm.at[ib.at[pl.ds(off + CHUNK_A, CHUNK_B)]],
                         rows_bufs[p].at[pl.ds(CHUNK_A, CHUNK_B)], gsems[p])

    def wait_bag(p):
        pltpu.make_async_copy(table_hbm.at[pl.ds(0, SEQ)], rows_bufs[p],
                              gsems[p]).wait()

    def accumulate(jl, p):
        rows = rows_bufs[p]

        def acc_step(k, carry):
            j = k * ACC_UNROLL
            for u in range(ACC_UNROLL):
                carry = tuple(
                    carry[c] + rows[j + u, pl.ds(c * 16, 16)]
                    for c in range(4))
            return carry

        zeros = jnp.zeros((16,), jnp.float32)
        acc = lax.fori_loop(0, SEQ // ACC_UNROLL, acc_step,
                            (zeros, zeros, zeros, zeros))
        for c in range(4):
            out_v[pl.ds(jl * EMBED_DIM + c * 16, 16)] = acc[c] * inv_n

    fire_idx(0)
    for b in range(NUM_BLOCKS):
        wait_idx(b)
        fire_bag(b, 0, 0)
        if b + 1 < NUM_BLOCKS:
            fire_idx(b + 1)

        def pair_body(t, _, b=b):
            jl0 = 2 * t
            wait_bag(0)
            fire_bag(b, jl0 + 1, 1)
            accumulate(jl0, 0)
            wait_bag(1)
            fire_bag(b, jl0 + 2, 0)
            accumulate(jl0 + 1, 1)
            return 0

        lax.fori_loop(0, IBLK // 2 - 1, pair_body, 0)
        wait_bag(0)
        fire_bag(b, IBLK - 1, 1)
        accumulate(IBLK - 2, 0)
        wait_bag(1)
        accumulate(IBLK - 1, 1)
        pltpu.sync_copy(
            out_v,
            out_hbm.at[pl.ds((base + b * IBLK) * EMBED_DIM,
                             IBLK * EMBED_DIM)])


@jax.jit
def _embed_mean(input_ids, emb_table):
    mesh = plsc.VectorSubcoreMesh(
        core_axis_name="c", subcore_axis_name="s",
        num_cores=NUM_CORES, num_subcores=NUM_SUBCORES)
    out = pl.kernel(
        _embed_mean_body,
        out_type=jax.ShapeDtypeStruct((BATCH * EMBED_DIM,), jnp.float32),
        mesh=mesh,
        scratch_types=[
            pltpu.VMEM((IBLK * SEQ,), jnp.int32),
            pltpu.VMEM((IBLK * SEQ,), jnp.int32),
            pltpu.VMEM((SEQ, EMBED_DIM), jnp.float32),
            pltpu.VMEM((SEQ, EMBED_DIM), jnp.float32),
            pltpu.VMEM((IBLK * EMBED_DIM,), jnp.float32),
            pltpu.SemaphoreType.DMA,
            pltpu.SemaphoreType.DMA,
            pltpu.SemaphoreType.DMA,
            pltpu.SemaphoreType.DMA,
        ],
        compiler_params=pltpu.CompilerParams(use_tc_tiling_on_sc=False),
    )(input_ids.reshape(-1), emb_table)
    return out.reshape(BATCH, EMBED_DIM)


MLP_BLOCK = 1024
OUT_PAD = 128


def _mlp_body(x_ref, w1_ref, b1_ref, w2_ref, b2_ref, wa_ref, ba_ref,
              wc_ref, bc_ref, out_ref):
    x = x_ref[...]
    h = jnp.maximum(jnp.dot(x, w1_ref[...],
                            preferred_element_type=jnp.float32) + b1_ref[...], 0.0)
    h = jnp.maximum(jnp.dot(h, w2_ref[...],
                            preferred_element_type=jnp.float32) + b2_ref[...], 0.0)
    gate = jax.nn.sigmoid(jnp.dot(h, wa_ref[...],
                                  preferred_element_type=jnp.float32) + ba_ref[...])
    h = h * gate
    out_ref[...] = jnp.dot(h, wc_ref[...],
                           preferred_element_type=jnp.float32) + bc_ref[...]


@jax.jit
def _mlp(x, W1, b1, W2, b2, Wa, ba, Wc, bc):
    h1 = W1.shape[1]
    h2 = W2.shape[1]
    ncls = Wc.shape[1]
    Wc_pad = jnp.zeros((h2, OUT_PAD), jnp.float32).at[:, :ncls].set(Wc)
    bc_pad = jnp.zeros((1, OUT_PAD), jnp.float32).at[0, :ncls].set(bc)
    grid = BATCH // MLP_BLOCK
    full = lambda shape: pl.BlockSpec(shape, lambda i: (0, 0))
    out = pl.pallas_call(
        _mlp_body,
        grid=(grid,),
        in_specs=[
            pl.BlockSpec((MLP_BLOCK, EMBED_DIM), lambda i: (i, 0)),
            full((EMBED_DIM, h1)),
            full((1, h1)),
            full((h1, h2)),
            full((1, h2)),
            full((h2, h2)),
            full((1, h2)),
            full((h2, OUT_PAD)),
            full((1, OUT_PAD)),
        ],
        out_specs=pl.BlockSpec((MLP_BLOCK, OUT_PAD), lambda i: (i, 0)),
        out_shape=jax.ShapeDtypeStruct((BATCH, OUT_PAD), jnp.float32),
    )(x, W1, b1.reshape(1, -1), W2, b2.reshape(1, -1),
      Wa, ba.reshape(1, -1), Wc_pad, bc_pad)
    return out[:, :ncls]


def kernel(input_ids, emb_table, W1, b1, W2, b2, Wa, ba, Wc, bc):
    embedded = _embed_mean(input_ids, emb_table)
    return _mlp(embedded, W1, b1, W2, b2, Wa, ba, Wc, bc)


# speedup vs baseline: 2.7999x; 1.0001x over previous
import functools

import jax
import jax.numpy as jnp
from jax import lax
from jax.experimental import pallas as pl
from jax.experimental.pallas import tpu as pltpu
from jax.experimental.pallas import tpu_sc as plsc

VOCAB = 1000000
EMBED_DIM = 64
BATCH = 16384
SEQ = 200

NUM_CORES = 2
NUM_SUBCORES = 16
NUM_WORKERS = NUM_CORES * NUM_SUBCORES
BAGS_PER_WORKER = BATCH // NUM_WORKERS
IBLK = 64
NUM_BLOCKS = BAGS_PER_WORKER // IBLK
CHUNK_A = 128
CHUNK_B = SEQ - CHUNK_A
ACC_UNROLL = 8


def _embed_mean_body(idx_hbm, table_hbm, out_hbm,
                     idx0, idx1, rows0, rows1, out_v,
                     isem0, isem1, gsem0, gsem1):
    wid = lax.axis_index("s") * NUM_CORES + lax.axis_index("c")
    base = wid * BAGS_PER_WORKER
    inv_n = jnp.float32(1.0 / SEQ)
    idx_bufs = (idx0, idx1)
    isems = (isem0, isem1)
    rows_bufs = (rows0, rows1)
    gsems = (gsem0, gsem1)

    def fire_idx(b):
        pltpu.async_copy(
            idx_hbm.at[pl.ds(base + b * IBLK, IBLK), :],
            idx_bufs[b % 2], isems[b % 2])

    def wait_idx(b):
        pltpu.make_async_copy(
            idx_hbm.at[pl.ds(0, IBLK), :], idx_bufs[b % 2],
            isems[b % 2]).wait()

    def fire_bag(b, jl, p):
        ib = idx_bufs[b % 2]
        pltpu.async_copy(table_hbm.at[ib.at[jl, pl.ds(0, CHUNK_A)]],
                         rows_bufs[p].at[pl.ds(0, CHUNK_A)], gsems[p])
        pltpu.async_copy(table_hbm.at[ib.at[jl, pl.ds(CHUNK_A, CHUNK_B)]],
                         rows_bufs[p].at[pl.ds(CHUNK_A, CHUNK_B)], gsems[p])

    def wait_bag(p):
        pltpu.make_async_copy(table_hbm.at[pl.ds(0, SEQ)], rows_bufs[p],
                              gsems[p]).wait()

    def accumulate(jl, p):
        rows = rows_bufs[p]

        def acc_step(k, carry):
            j = k * ACC_UNROLL
            for u in range(ACC_UNROLL):
                carry = tuple(
                    carry[c] + rows[j + u, pl.ds(c * 16, 16)]
                    for c in range(4))
            return carry

        zeros = jnp.zeros((16,), jnp.float32)
        acc = lax.fori_loop(0, SEQ // ACC_UNROLL, acc_step,
                            (zeros, zeros, zeros, zeros))
        for c in range(4):
            out_v[pl.ds(jl * EMBED_DIM + c * 16, 16)] = acc[c] * inv_n

    fire_idx(0)
    for b in range(NUM_BLOCKS):
        wait_idx(b)
        fire_bag(b, 0, 0)
        if b + 1 < NUM_BLOCKS:
            fire_idx(b + 1)

        def pair_body(t, _, b=b):
            jl0 = 2 * t
            wait_bag(0)
            fire_bag(b, jl0 + 1, 1)
            accumulate(jl0, 0)
            wait_bag(1)
            fire_bag(b, jl0 + 2, 0)
            accumulate(jl0 + 1, 1)
            return 0

        lax.fori_loop(0, IBLK // 2 - 1, pair_body, 0)
        wait_bag(0)
        fire_bag(b, IBLK - 1, 1)
        accumulate(IBLK - 2, 0)
        wait_bag(1)
        accumulate(IBLK - 1, 1)
        pltpu.sync_copy(
            out_v,
            out_hbm.at[pl.ds((base + b * IBLK) * EMBED_DIM,
                             IBLK * EMBED_DIM)])


@jax.jit
def _embed_mean(input_ids, emb_table):
    mesh = plsc.VectorSubcoreMesh(
        core_axis_name="c", subcore_axis_name="s",
        num_cores=NUM_CORES, num_subcores=NUM_SUBCORES)
    out = pl.kernel(
        _embed_mean_body,
        out_type=jax.ShapeDtypeStruct((BATCH * EMBED_DIM,), jnp.float32),
        mesh=mesh,
        scratch_types=[
            pltpu.VMEM((IBLK, SEQ), jnp.int32),
            pltpu.VMEM((IBLK, SEQ), jnp.int32),
            pltpu.VMEM((SEQ, EMBED_DIM), jnp.float32),
            pltpu.VMEM((SEQ, EMBED_DIM), jnp.float32),
            pltpu.VMEM((IBLK * EMBED_DIM,), jnp.float32),
            pltpu.SemaphoreType.DMA,
            pltpu.SemaphoreType.DMA,
            pltpu.SemaphoreType.DMA,
            pltpu.SemaphoreType.DMA,
        ],
        compiler_params=pltpu.CompilerParams(use_tc_tiling_on_sc=False),
    )(input_ids, emb_table)
    return out.reshape(BATCH, EMBED_DIM)


MLP_BLOCK = 1024
OUT_PAD = 128


def _mlp_body(x_ref, w1_ref, b1_ref, w2_ref, b2_ref, wa_ref, ba_ref,
              wc_ref, bc_ref, out_ref):
    x = x_ref[...]
    h = jnp.maximum(jnp.dot(x, w1_ref[...],
                            preferred_element_type=jnp.float32) + b1_ref[...], 0.0)
    h = jnp.maximum(jnp.dot(h, w2_ref[...],
                            preferred_element_type=jnp.float32) + b2_ref[...], 0.0)
    gate = jax.nn.sigmoid(jnp.dot(h, wa_ref[...],
                                  preferred_element_type=jnp.float32) + ba_ref[...])
    h = h * gate
    out_ref[...] = jnp.dot(h, wc_ref[...],
                           preferred_element_type=jnp.float32) + bc_ref[...]


@jax.jit
def _mlp(x, W1, b1, W2, b2, Wa, ba, Wc, bc):
    h1 = W1.shape[1]
    h2 = W2.shape[1]
    ncls = Wc.shape[1]
    Wc_pad = jnp.zeros((h2, OUT_PAD), jnp.float32).at[:, :ncls].set(Wc)
    bc_pad = jnp.zeros((1, OUT_PAD), jnp.float32).at[0, :ncls].set(bc)
    grid = BATCH // MLP_BLOCK
    full = lambda shape: pl.BlockSpec(shape, lambda i: (0, 0))
    out = pl.pallas_call(
        _mlp_body,
        grid=(grid,),
        in_specs=[
            pl.BlockSpec((MLP_BLOCK, EMBED_DIM), lambda i: (i, 0)),
            full((EMBED_DIM, h1)),
            full((1, h1)),
            full((h1, h2)),
            full((1, h2)),
            full((h2, h2)),
            full((1, h2)),
            full((h2, OUT_PAD)),
            full((1, OUT_PAD)),
        ],
        out_specs=pl.BlockSpec((MLP_BLOCK, OUT_PAD), lambda i: (i, 0)),
        out_shape=jax.ShapeDtypeStruct((BATCH, OUT_PAD), jnp.float32),
    )(x, W1, b1.reshape(1, -1), W2, b2.reshape(1, -1),
      Wa, ba.reshape(1, -1), Wc_pad, bc_pad)
    return out[:, :ncls]


def kernel(input_ids, emb_table, W1, b1, W2, b2, Wa, ba, Wc, bc):
    embedded = _embed_mean(input_ids, emb_table)
    return _mlp(embedded, W1, b1, W2, b2, Wa, ba, Wc, bc)


# speedup vs baseline: 2.8253x; 1.0091x over previous
import jax
import jax.numpy as jnp
from jax import lax
from jax.experimental import pallas as pl
from jax.experimental.pallas import tpu as pltpu
from jax.experimental.pallas import tpu_sc as plsc

VOCAB = 1000000
EMBED_DIM = 64
BATCH = 16384
SEQ = 200

NUM_CORES = 2
NUM_SUBCORES = 16
NUM_WORKERS = NUM_CORES * NUM_SUBCORES
BAGS_PER_WORKER = BATCH // NUM_WORKERS
IBLK = 64
NUM_BLOCKS = BAGS_PER_WORKER // IBLK
CHUNK_A = 128
CHUNK_B = SEQ - CHUNK_A
ACC_UNROLL = 8

REPACK_BLK = 512


def _repack_body(ids_ref, a_ref, b_ref):
    x = ids_ref[...]
    a_ref[...] = x[:, :CHUNK_A]
    b_ref[...] = jnp.concatenate(
        [x[:, CHUNK_A:SEQ],
         jnp.zeros((REPACK_BLK, 128 - CHUNK_B), jnp.int32)], axis=1)


def _repack(input_ids):
    return pl.pallas_call(
        _repack_body,
        grid=(BATCH // REPACK_BLK,),
        in_specs=[pl.BlockSpec((REPACK_BLK, SEQ), lambda i: (i, 0))],
        out_specs=[
            pl.BlockSpec((REPACK_BLK, 128), lambda i: (i, 0)),
            pl.BlockSpec((REPACK_BLK, 128), lambda i: (i, 0)),
        ],
        out_shape=[
            jax.ShapeDtypeStruct((BATCH, 128), jnp.int32),
            jax.ShapeDtypeStruct((BATCH, 128), jnp.int32),
        ],
    )(input_ids)


def _embed_mean_body(ida_hbm, idb_hbm, table_hbm, out_hbm,
                     ida0, ida1, idb0, idb1, rows0, rows1, out_v,
                     isem0, isem1, gsem0, gsem1):
    wid = lax.axis_index("s") * NUM_CORES + lax.axis_index("c")
    base = wid * BAGS_PER_WORKER
    inv_n = jnp.float32(1.0 / SEQ)
    ida_bufs = (ida0, ida1)
    idb_bufs = (idb0, idb1)
    isems = (isem0, isem1)
    rows_bufs = (rows0, rows1)
    gsems = (gsem0, gsem1)

    def fire_idx(b):
        pltpu.async_copy(
            ida_hbm.at[pl.ds(base + b * IBLK, IBLK), :],
            ida_bufs[b % 2], isems[b % 2])
        pltpu.async_copy(
            idb_hbm.at[pl.ds(base + b * IBLK, IBLK), :],
            idb_bufs[b % 2], isems[b % 2])

    def wait_idx(b):
        pltpu.make_async_copy(
            ida_hbm.at[pl.ds(0, IBLK), :], ida_bufs[b % 2],
            isems[b % 2]).wait()
        pltpu.make_async_copy(
            idb_hbm.at[pl.ds(0, IBLK), :], idb_bufs[b % 2],
            isems[b % 2]).wait()

    def fire_bag(b, jl, p):
        ia = ida_bufs[b % 2]
        ib = idb_bufs[b % 2]
        pltpu.async_copy(table_hbm.at[ia.at[jl, pl.ds(0, CHUNK_A)]],
                         rows_bufs[p].at[pl.ds(0, CHUNK_A)], gsems[p])
        pltpu.async_copy(table_hbm.at[ib.at[jl, pl.ds(0, CHUNK_B)]],
                         rows_bufs[p].at[pl.ds(CHUNK_A, CHUNK_B)], gsems[p])

    def wait_bag(p):
        pltpu.make_async_copy(table_hbm.at[pl.ds(0, SEQ)], rows_bufs[p],
                              gsems[p]).wait()

    def accumulate(p, row, half):
        rows = rows_bufs[p]

        def acc_step(k, carry):
            j = k * ACC_UNROLL
            for u in range(ACC_UNROLL):
                carry = tuple(
                    carry[c] + rows[j + u, pl.ds(c * 16, 16)]
                    for c in range(4))
            return carry

        zeros = jnp.zeros((16,), jnp.float32)
        acc = lax.fori_loop(0, SEQ // ACC_UNROLL, acc_step,
                            (zeros, zeros, zeros, zeros))
        for c in range(4):
            out_v[row, pl.ds(half * EMBED_DIM + c * 16, 16)] = acc[c] * inv_n

    fire_idx(0)
    for b in range(NUM_BLOCKS):
        wait_idx(b)
        fire_bag(b, 0, 0)
        if b + 1 < NUM_BLOCKS:
            fire_idx(b + 1)

        def pair_body(t, _, b=b):
            jl0 = 2 * t
            wait_bag(0)
            fire_bag(b, jl0 + 1, 1)
            accumulate(0, t, 0)
            wait_bag(1)
            fire_bag(b, jl0 + 2, 0)
            accumulate(1, t, 1)
            return 0

        lax.fori_loop(0, IBLK // 2 - 1, pair_body, 0)
        wait_bag(0)
        fire_bag(b, IBLK - 1, 1)
        accumulate(0, IBLK // 2 - 1, 0)
        wait_bag(1)
        accumulate(1, IBLK // 2 - 1, 1)
        pltpu.sync_copy(
            out_v,
            out_hbm.at[pl.ds((base + b * IBLK) // 2, IBLK // 2), :])


def _embed_mean(ida, idb, emb_table):
    mesh = plsc.VectorSubcoreMesh(
        core_axis_name="c", subcore_axis_name="s",
        num_cores=NUM_CORES, num_subcores=NUM_SUBCORES)
    return pl.kernel(
        _embed_mean_body,
        out_type=jax.ShapeDtypeStruct((BATCH // 2, 128), jnp.float32),
        mesh=mesh,
        scratch_types=[
            pltpu.VMEM((IBLK, 128), jnp.int32),
            pltpu.VMEM((IBLK, 128), jnp.int32),
            pltpu.VMEM((IBLK, 128), jnp.int32),
            pltpu.VMEM((IBLK, 128), jnp.int32),
            pltpu.VMEM((SEQ, EMBED_DIM), jnp.float32),
            pltpu.VMEM((SEQ, EMBED_DIM), jnp.float32),
            pltpu.VMEM((IBLK // 2, 128), jnp.float32),
            pltpu.SemaphoreType.DMA,
            pltpu.SemaphoreType.DMA,
            pltpu.SemaphoreType.DMA,
            pltpu.SemaphoreType.DMA,
        ],
        compiler_params=pltpu.CompilerParams(use_tc_tiling_on_sc=False),
    )(ida, idb, emb_table)


MLP_BLOCK = 512


def _mlp_body(x_ref, w1_ref, b1_ref, w2_ref, b2_ref, wa_ref, ba_ref,
              wc_ref, bc_ref, out_ref):
    x = x_ref[...]
    h = jnp.maximum(jnp.dot(x, w1_ref[...],
                            preferred_element_type=jnp.float32) + b1_ref[...], 0.0)
    h = jnp.maximum(jnp.dot(h, w2_ref[...],
                            preferred_element_type=jnp.float32) + b2_ref[...], 0.0)
    gate = jax.nn.sigmoid(jnp.dot(h, wa_ref[...],
                                  preferred_element_type=jnp.float32) + ba_ref[...])
    h = h * gate
    out_ref[...] = jnp.dot(h, wc_ref[...],
                           preferred_element_type=jnp.float32) + bc_ref[...]


def _mlp_packed(x, W1, b1, W2, b2, Wa, ba, Wc, bc):
    h1 = W1.shape[1]
    h2 = W2.shape[1]
    ncls = Wc.shape[1]
    W1p = jnp.zeros((128, 2 * h1), jnp.float32)
    W1p = W1p.at[:EMBED_DIM, :h1].set(W1).at[EMBED_DIM:, h1:].set(W1)
    b1p = jnp.concatenate([b1, b1]).reshape(1, -1)
    W2p = jnp.zeros((2 * h1, 2 * h2), jnp.float32)
    W2p = W2p.at[:h1, :h2].set(W2).at[h1:, h2:].set(W2)
    b2p = jnp.concatenate([b2, b2]).reshape(1, -1)
    Wap = jnp.zeros((2 * h2, 2 * h2), jnp.float32)
    Wap = Wap.at[:h2, :h2].set(Wa).at[h2:, h2:].set(Wa)
    bap = jnp.concatenate([ba, ba]).reshape(1, -1)
    Wcp = jnp.zeros((2 * h2, 128), jnp.float32)
    Wcp = Wcp.at[:h2, :ncls].set(Wc).at[h2:, ncls:2 * ncls].set(Wc)
    bcp = jnp.zeros((1, 128), jnp.float32)
    bcp = bcp.at[0, :ncls].set(bc).at[0, ncls:2 * ncls].set(bc)

    grid = (BATCH // 2) // MLP_BLOCK
    full = lambda shape: pl.BlockSpec(shape, lambda i: (0, 0))
    out = pl.pallas_call(
        _mlp_body,
        grid=(grid,),
        in_specs=[
            pl.BlockSpec((MLP_BLOCK, 128), lambda i: (i, 0)),
            full((128, 2 * h1)),
            full((1, 2 * h1)),
            full((2 * h1, 2 * h2)),
            full((1, 2 * h2)),
            full((2 * h2, 2 * h2)),
            full((1, 2 * h2)),
            full((2 * h2, 128)),
            full((1, 128)),
        ],
        out_specs=pl.BlockSpec((MLP_BLOCK, 128), lambda i: (i, 0)),
        out_shape=jax.ShapeDtypeStruct((BATCH // 2, 128), jnp.float32),
    )(x, W1p, b1p, W2p, b2p, Wap, bap, Wcp, bcp)
    return out[:, :2 * ncls].reshape(BATCH, ncls)


@jax.jit
def kernel(input_ids, emb_table, W1, b1, W2, b2, Wa, ba, Wc, bc):
    ida, idb = _repack(input_ids)
    packed = _embed_mean(ida, idb, emb_table)
    return _mlp_packed(packed, W1, b1, W2, b2, Wa, ba, Wc, bc)


# speedup vs baseline: 2.8665x; 1.0146x over previous
import jax
import jax.numpy as jnp
from jax import lax
from jax.experimental import pallas as pl
from jax.experimental.pallas import tpu as pltpu
from jax.experimental.pallas import tpu_sc as plsc

VOCAB = 1000000
EMBED_DIM = 64
BATCH = 16384
SEQ = 200

NUM_CORES = 2
NUM_SUBCORES = 16
NUM_WORKERS = NUM_CORES * NUM_SUBCORES
BAGS_PER_WORKER = BATCH // NUM_WORKERS
HALF = BAGS_PER_WORKER // 2
ACC_ROWS = BAGS_PER_WORKER // 2
PAIR_UNROLL = 4


def _embed_mean_body(idt_hbm, table_hbm, out_hbm,
                     idx0, idx1, rows0, rows1, acc,
                     isem0, isem1, gsem0, gsem1):
    wid = lax.axis_index("s") * NUM_CORES + lax.axis_index("c")
    cbase = wid * BAGS_PER_WORKER
    inv_n = jnp.float32(1.0 / SEQ)
    idx_bufs = (idx0, idx1)
    isems = (isem0, isem1)
    rows_bufs = (rows0, rows1)
    gsems = (gsem0, gsem1)

    def fire_idx(j, k):
        pltpu.async_copy(
            idt_hbm.at[j, pl.ds(cbase, BAGS_PER_WORKER)],
            idx_bufs[k], isems[k])

    def wait_idx(k):
        pltpu.make_async_copy(
            idt_hbm.at[0, pl.ds(0, BAGS_PER_WORKER)], idx_bufs[k],
            isems[k]).wait()

    def fire_g(h, k):
        ib = idx_bufs[k]
        pltpu.async_copy(table_hbm.at[ib.at[pl.ds(h * HALF, 128)]],
                         rows_bufs[h].at[pl.ds(0, 128)], gsems[h])
        pltpu.async_copy(table_hbm.at[ib.at[pl.ds(h * HALF + 128, 128)]],
                         rows_bufs[h].at[pl.ds(128, 128)], gsems[h])

    def wait_g(h):
        pltpu.make_async_copy(table_hbm.at[pl.ds(0, HALF)], rows_bufs[h],
                              gsems[h]).wait()

    def accumulate(h, first):
        rows = rows_bufs[h]
        rbase = h * (HALF // 2)

        def pair_step(s0, _):
            for q in range(PAIR_UNROLL):
                s = s0 * PAIR_UNROLL + q
                for u in range(2):
                    for c in range(4):
                        v = rows[2 * s + u, pl.ds(c * 16, 16)]
                        dst = acc.at[rbase + s,
                                     pl.ds(u * EMBED_DIM + c * 16, 16)]
                        if first:
                            dst[...] = v
                        else:
                            plsc.addupdate(dst, v)
            return 0

        lax.fori_loop(0, (HALF // 2) // PAIR_UNROLL, pair_step, 0)

    fire_idx(0, 0)
    wait_idx(0)
    fire_g(0, 0)
    fire_g(1, 0)
    fire_idx(1, 1)
    wait_idx(1)
    wait_g(0)
    accumulate(0, True)
    fire_g(0, 1)
    wait_g(1)
    accumulate(1, True)
    fire_g(1, 1)
    fire_idx(2, 0)

    def body(u, _):
        wait_idx(0)
        wait_g(0)
        accumulate(0, False)
        fire_g(0, 0)
        wait_g(1)
        accumulate(1, False)
        fire_g(1, 0)
        fire_idx(2 * u + 3, 1)
        wait_idx(1)
        wait_g(0)
        accumulate(0, False)
        fire_g(0, 1)
        wait_g(1)
        accumulate(1, False)
        fire_g(1, 1)
        fire_idx(jnp.minimum(2 * u + 4, SEQ - 1), 0)
        return 0

    lax.fori_loop(0, (SEQ - 2) // 2, body, 0)
    wait_g(0)
    accumulate(0, False)
    wait_g(1)
    accumulate(1, False)
    wait_idx(0)

    def scale_step(r, _):
        for c in range(8):
            acc[r, pl.ds(c * 16, 16)] = acc[r, pl.ds(c * 16, 16)] * inv_n
        return 0

    lax.fori_loop(0, ACC_ROWS, scale_step, 0)
    pltpu.sync_copy(acc, out_hbm.at[pl.ds(wid * ACC_ROWS, ACC_ROWS), :])


def _embed_mean(input_ids, emb_table):
    mesh = plsc.VectorSubcoreMesh(
        core_axis_name="c", subcore_axis_name="s",
        num_cores=NUM_CORES, num_subcores=NUM_SUBCORES)
    return pl.kernel(
        _embed_mean_body,
        out_type=jax.ShapeDtypeStruct((BATCH // 2, 128), jnp.float32),
        mesh=mesh,
        scratch_types=[
            pltpu.VMEM((BAGS_PER_WORKER,), jnp.int32),
            pltpu.VMEM((BAGS_PER_WORKER,), jnp.int32),
            pltpu.VMEM((HALF, EMBED_DIM), jnp.float32),
            pltpu.VMEM((HALF, EMBED_DIM), jnp.float32),
            pltpu.VMEM((ACC_ROWS, 128), jnp.float32),
            pltpu.SemaphoreType.DMA,
            pltpu.SemaphoreType.DMA,
            pltpu.SemaphoreType.DMA,
            pltpu.SemaphoreType.DMA,
        ],
        compiler_params=pltpu.CompilerParams(use_tc_tiling_on_sc=False),
    )(input_ids.T, emb_table)


MLP_BLOCK = 512


def _mlp_body(x_ref, w1_ref, b1_ref, w2_ref, b2_ref, wa_ref, ba_ref,
              wc_ref, bc_ref, out_ref):
    x = x_ref[...]
    h = jnp.maximum(jnp.dot(x, w1_ref[...],
                            preferred_element_type=jnp.float32) + b1_ref[...], 0.0)
    h = jnp.maximum(jnp.dot(h, w2_ref[...],
                            preferred_element_type=jnp.float32) + b2_ref[...], 0.0)
    gate = jax.nn.sigmoid(jnp.dot(h, wa_ref[...],
                                  preferred_element_type=jnp.float32) + ba_ref[...])
    h = h * gate
    out_ref[...] = jnp.dot(h, wc_ref[...],
                           preferred_element_type=jnp.float32) + bc_ref[...]


def _mlp_packed(x, W1, b1, W2, b2, Wa, ba, Wc, bc):
    h1 = W1.shape[1]
    h2 = W2.shape[1]
    ncls = Wc.shape[1]
    W1p = jnp.zeros((128, 2 * h1), jnp.float32)
    W1p = W1p.at[:EMBED_DIM, :h1].set(W1).at[EMBED_DIM:, h1:].set(W1)
    b1p = jnp.concatenate([b1, b1]).reshape(1, -1)
    W2p = jnp.zeros((2 * h1, 2 * h2), jnp.float32)
    W2p = W2p.at[:h1, :h2].set(W2).at[h1:, h2:].set(W2)
    b2p = jnp.concatenate([b2, b2]).reshape(1, -1)
    Wap = jnp.zeros((2 * h2, 2 * h2), jnp.float32)
    Wap = Wap.at[:h2, :h2].set(Wa).at[h2:, h2:].set(Wa)
    bap = jnp.concatenate([ba, ba]).reshape(1, -1)
    Wcp = jnp.zeros((2 * h2, 128), jnp.float32)
    Wcp = Wcp.at[:h2, :ncls].set(Wc).at[h2:, ncls:2 * ncls].set(Wc)
    bcp = jnp.zeros((1, 128), jnp.float32)
    bcp = bcp.at[0, :ncls].set(bc).at[0, ncls:2 * ncls].set(bc)

    grid = (BATCH // 2) // MLP_BLOCK
    full = lambda shape: pl.BlockSpec(shape, lambda i: (0, 0))
    out = pl.pallas_call(
        _mlp_body,
        grid=(grid,),
        in_specs=[
            pl.BlockSpec((MLP_BLOCK, 128), lambda i: (i, 0)),
            full((128, 2 * h1)),
            full((1, 2 * h1)),
            full((2 * h1, 2 * h2)),
            full((1, 2 * h2)),
            full((2 * h2, 2 * h2)),
            full((1, 2 * h2)),
            full((2 * h2, 128)),
            full((1, 128)),
        ],
        out_specs=pl.BlockSpec((MLP_BLOCK, 128), lambda i: (i, 0)),
        out_shape=jax.ShapeDtypeStruct((BATCH // 2, 128), jnp.float32),
    )(x, W1p, b1p, W2p, b2p, Wap, bap, Wcp, bcp)
    return out[:, :2 * ncls].reshape(BATCH, ncls)


@jax.jit
def kernel(input_ids, emb_table, W1, b1, W2, b2, Wa, ba, Wc, bc):
    packed = _embed_mean(input_ids, emb_table)
    return _mlp_packed(packed, W1, b1, W2, b2, Wa, ba, Wc, bc)
